# Initial kernel scaffold; baseline (speedup 1.0000x reference)
import jax
import jax.numpy as jnp
from jax.experimental import pallas as pl


def kernel(x, edge_index, orders, dist, W_em, b_em, W1, b1, W2, b2, W3, b3, W_out, b_out):
    raise NotImplementedError("write your pallas kernel here")


# speedup vs baseline: 1.6231x; 1.6231x over previous
import jax
import jax.numpy as jnp
from jax.experimental import pallas as pl

_N = 100000
_K = 10
_F32 = jnp.float32


def _wave_body(x_ref, cnt_ref, sx_ref, sd_ref, a_ref, b_ref, bb_ref, c_ref, o_ref):
    x = x_ref[...]
    upd = cnt_ref[...] * (
        jnp.dot(x, a_ref[...], preferred_element_type=_F32) + b_ref[...]
    )
    upd += jnp.dot(sx_ref[...], bb_ref[...], preferred_element_type=_F32)
    upd += jnp.dot(sd_ref[...], c_ref[...], preferred_element_type=_F32)
    o_ref[...] = x + upd


def _matmul_body(x_ref, w_ref, o_ref):
    o_ref[...] = jnp.dot(x_ref[...], w_ref[...], preferred_element_type=_F32)


def _combine_body(x_ref, m_ref, mask_ref, a_ref, b_ref, o_ref):
    z = jnp.dot(x_ref[...], a_ref[...], preferred_element_type=_F32) + b_ref[...]
    o_ref[...] = jnp.where(mask_ref[...] > 0, z + m_ref[...], 0.0)


def _combine_out_body(x_ref, m_ref, mask_ref, a_ref, b_ref, wo_ref, bo_ref, o_ref):
    z = jnp.dot(x_ref[...], a_ref[...], preferred_element_type=_F32) + b_ref[...]
    h = jnp.where(mask_ref[...] > 0, z + m_ref[...], 0.0)
    o_ref[...] = jnp.dot(h, wo_ref[...], preferred_element_type=_F32) + bo_ref[...]


_BN = 2000


def _call(body, out_dim, *args):
    in_specs = []
    for a in args:
        if a.shape[0] == _N:
            in_specs.append(
                pl.BlockSpec((_BN, a.shape[1]), lambda i: (i, 0))
            )
        else:
            in_specs.append(
                pl.BlockSpec(a.shape, lambda i: (0, 0))
            )
    return pl.pallas_call(
        body,
        grid=(_N // _BN,),
        in_specs=in_specs,
        out_specs=pl.BlockSpec((_BN, out_dim), lambda i: (i, 0)),
        out_shape=jax.ShapeDtypeStruct((_N, out_dim), _F32),
    )(*args)


@jax.jit
def kernel(x, edge_index, orders, dist, W_em, b_em, W1, b1, W2, b2, W3, b3, W_out, b_out):
    n = x.shape[0]
    src = edge_index[0]
    dst = edge_index[1]

    A_em = W_em[:_K] - W_em[_K : 2 * _K]
    B_em = W_em[_K : 2 * _K]
    C_em = W_em[2 * _K :]
    b_em2 = b_em.reshape(1, -1)
    for i in range(orders.shape[0]):
        order = orders[i]
        e0 = jnp.take(edge_index[0], order)
        e1 = jnp.take(edge_index[1], order)
        cnt = jax.ops.segment_sum(
            jnp.ones(order.shape, _F32), e0, num_segments=n
        ).reshape(n, 1)
        sx = jax.ops.segment_sum(jnp.take(x, e1, axis=0), e0, num_segments=n)
        sd = jax.ops.segment_sum(jnp.take(dist, order, axis=0), e0, num_segments=n)
        x = _call(_wave_body, _K, x, cnt, sx, sd, A_em, b_em2, B_em, C_em)

    deg = jax.ops.segment_sum(
        jnp.ones(dst.shape, _F32), dst, num_segments=n
    ).reshape(n, 1)

    for layer, (W, b) in enumerate(((W1, b1), (W2, b2), (W3, b3))):
        A = W[:_K] - W[_K:]
        B = W[_K:]
        y = _call(_matmul_body, _K, x, B)
        m = jax.ops.segment_max(jnp.take(y, src, axis=0), dst, num_segments=n)
        if layer < 2:
            x = _call(_combine_body, _K, x, m, deg, A, b.reshape(1, -1))
        else:
            x = _call(
                _combine_out_body,
                W_out.shape[1],
                x, m, deg, A, b.reshape(1, -1), W_out, b_out.reshape(1, -1),
            )
    return x


# speedup vs baseline: 1.6881x; 1.0401x over previous
import jax
import jax.numpy as jnp
from jax.experimental import pallas as pl

_N = 100000
_K = 10
_F32 = jnp.float32


def _wave_body(x_ref, cnt_ref, sx_ref, sd_ref, a_ref, b_ref, bb_ref, c_ref, o_ref):
    x = x_ref[...]
    upd = cnt_ref[...] * (
        jnp.dot(x, a_ref[...], preferred_element_type=_F32) + b_ref[...]
    )
    upd += jnp.dot(sx_ref[...], bb_ref[...], preferred_element_type=_F32)
    upd += jnp.dot(sd_ref[...], c_ref[...], preferred_element_type=_F32)
    o_ref[...] = x + upd


def _matmul_body(x_ref, w_ref, o_ref):
    o_ref[...] = jnp.dot(x_ref[...], w_ref[...], preferred_element_type=_F32)


def _combine_body(x_ref, m_ref, mask_ref, a_ref, b_ref, o_ref):
    z = jnp.dot(x_ref[...], a_ref[...], preferred_element_type=_F32) + b_ref[...]
    o_ref[...] = jnp.where(mask_ref[...] > 0, z + m_ref[...], 0.0)


def _combine_out_body(x_ref, m_ref, mask_ref, a_ref, b_ref, wo_ref, bo_ref, o_ref):
    z = jnp.dot(x_ref[...], a_ref[...], preferred_element_type=_F32) + b_ref[...]
    h = jnp.where(mask_ref[...] > 0, z + m_ref[...], 0.0)
    o_ref[...] = jnp.dot(h, wo_ref[...], preferred_element_type=_F32) + bo_ref[...]


_BN = 2000


def _call(body, out_dim, *args):
    in_specs = []
    for a in args:
        if a.shape[0] == _N:
            in_specs.append(
                pl.BlockSpec((_BN, a.shape[1]), lambda i: (i, 0))
            )
        else:
            in_specs.append(
                pl.BlockSpec(a.shape, lambda i: (0, 0))
            )
    return pl.pallas_call(
        body,
        grid=(_N // _BN,),
        in_specs=in_specs,
        out_specs=pl.BlockSpec((_BN, out_dim), lambda i: (i, 0)),
        out_shape=jax.ShapeDtypeStruct((_N, out_dim), _F32),
    )(*args)


@jax.jit
def kernel(x, edge_index, orders, dist, W_em, b_em, W1, b1, W2, b2, W3, b3, W_out, b_out):
    n = x.shape[0]
    src = edge_index[0]
    dst = edge_index[1]

    A_em = W_em[:_K] - W_em[_K : 2 * _K]
    B_em = W_em[_K : 2 * _K]
    C_em = W_em[2 * _K :]
    b_em2 = b_em.reshape(1, -1)
    for i in range(orders.shape[0]):
        order = orders[i]
        e0 = jnp.take(edge_index[0], order)
        e1 = jnp.take(edge_index[1], order)
        vals = jnp.concatenate(
            [
                jnp.take(x, e1, axis=0),
                jnp.take(dist, order, axis=0),
                jnp.ones((order.shape[0], 1), _F32),
            ],
            axis=1,
        )
        s = jax.ops.segment_sum(vals, e0, num_segments=n)
        sx = s[:, :_K]
        sd = s[:, _K : _K + 4]
        cnt = s[:, _K + 4 :]
        x = _call(_wave_body, _K, x, cnt, sx, sd, A_em, b_em2, B_em, C_em)

    deg = jax.ops.segment_sum(
        jnp.ones(dst.shape, _F32), dst, num_segments=n
    ).reshape(n, 1)

    for layer, (W, b) in enumerate(((W1, b1), (W2, b2), (W3, b3))):
        A = W[:_K] - W[_K:]
        B = W[_K:]
        y = _call(_matmul_body, _K, x, B)
        m = jax.ops.segment_max(jnp.take(y, src, axis=0), dst, num_segments=n)
        if layer < 2:
            x = _call(_combine_body, _K, x, m, deg, A, b.reshape(1, -1))
        else:
            x = _call(
                _combine_out_body,
                W_out.shape[1],
                x, m, deg, A, b.reshape(1, -1), W_out, b_out.reshape(1, -1),
            )
    return x
